# baseline (speedup 1.0000x reference)
import functools

import jax
import jax.numpy as jnp
from jax import lax
from jax.experimental import pallas as pl
from jax.experimental.pallas import tpu as pltpu
from jax.experimental.pallas import tpu_sc as plsc

D = 64
LANES = 16
NC, NS = 2, 16
NW = NC * NS
SCALE = 8.0

BATCH = 128
S = 8
CHUNK = S * BATCH


def _build(n_rows):
    assert n_rows % (NW * CHUNK) == 0
    rows_per_w = n_rows // NW
    n_chunks = rows_per_w // CHUNK
    batches_per_w = rows_per_w // BATCH

    mesh = plsc.VectorSubcoreMesh(core_axis_name="c", subcore_axis_name="s")

    @functools.partial(
        pl.kernel,
        out_type=jax.ShapeDtypeStruct((n_rows, D), jnp.float32),
        mesh=mesh,
        scratch_types=[
            pltpu.VMEM((S, BATCH), jnp.int32),
            pltpu.VMEM((CHUNK, D), jnp.float32),
            pltpu.SemaphoreType.DMA,
        ],
        compiler_params=pltpu.CompilerParams(use_tc_tiling_on_sc=False),
    )
    def body(idx_hbm, table_hbm, out_hbm, idx_v, rows_v, sem):
        wid = lax.axis_index("s") * NC + lax.axis_index("c")
        batch0 = wid * batches_per_w
        row0 = wid * rows_per_w

        def chunk_body(ci, carry):
            pltpu.sync_copy(idx_hbm.at[pl.ds(batch0 + ci * S, S)], idx_v)
            copies = []
            for j in range(S):
                copies.append(
                    pltpu.async_copy(
                        table_hbm.at[idx_v.at[j]],
                        rows_v.at[pl.ds(j * BATCH, BATCH)],
                        sem,
                    )
                )
            for c in copies:
                c.wait()

            def scale_row(r, carry2):
                for c in range(D // LANES):
                    sl = (r, pl.ds(c * LANES, LANES))
                    rows_v[sl] = rows_v[sl] * SCALE
                return carry2

            lax.fori_loop(0, CHUNK, scale_row, 0)
            pltpu.sync_copy(rows_v, out_hbm.at[pl.ds(row0 + ci * CHUNK, CHUNK)])
            return carry

        lax.fori_loop(0, n_chunks, chunk_body, 0)

    return body


def kernel(x, table):
    b, t = x.shape
    n_rows = b * t
    idx2d = x.astype(jnp.int32).reshape(n_rows // BATCH, BATCH)
    out = _build(n_rows)(idx2d, table)
    return out.reshape(b, t, D)


# speedup vs baseline: 1.1741x; 1.1741x over previous
import functools

import jax
import jax.numpy as jnp
from jax import lax
from jax.experimental import pallas as pl
from jax.experimental.pallas import tpu as pltpu
from jax.experimental.pallas import tpu_sc as plsc

D = 64
LANES = 16
NC, NS = 2, 16
NW = NC * NS
SCALE = 8.0

BATCH = 128
S = 4
CHUNK = S * BATCH
RU = 8


def _build(n_rows):
    assert n_rows % (NW * CHUNK) == 0
    rows_per_w = n_rows // NW
    n_chunks = rows_per_w // CHUNK
    batches_per_w = rows_per_w // BATCH
    assert n_chunks % 2 == 0 and n_chunks >= 4

    mesh = plsc.VectorSubcoreMesh(core_axis_name="c", subcore_axis_name="s")

    @functools.partial(
        pl.kernel,
        out_type=jax.ShapeDtypeStruct((n_rows, D), jnp.float32),
        mesh=mesh,
        scratch_types=[
            pltpu.VMEM((2, S, BATCH), jnp.int32),
            pltpu.VMEM((2, CHUNK, D), jnp.float32),
            pltpu.SemaphoreType.DMA,
            pltpu.SemaphoreType.DMA,
            pltpu.SemaphoreType.DMA,
            pltpu.SemaphoreType.DMA,
            pltpu.SemaphoreType.DMA,
            pltpu.SemaphoreType.DMA,
        ],
        compiler_params=pltpu.CompilerParams(use_tc_tiling_on_sc=False),
    )
    def body(idx_hbm, table_hbm, out_hbm, idx_v, rows_v,
             gsem0, gsem1, wsem0, wsem1, isem0, isem1):
        gsem = (gsem0, gsem1)
        wsem = (wsem0, wsem1)
        isem = (isem0, isem1)
        wid = lax.axis_index("s") * NC + lax.axis_index("c")
        batch0 = wid * batches_per_w
        row0 = wid * rows_per_w

        def fire_gathers(b, ci):
            for j in range(S):
                pltpu.async_copy(
                    table_hbm.at[idx_v.at[b, j]],
                    rows_v.at[b, pl.ds(j * BATCH, BATCH)],
                    gsem[b],
                )

        def wait_gathers(b):
            for j in range(S):
                pltpu.make_async_copy(
                    table_hbm.at[idx_v.at[b, j]],
                    rows_v.at[b, pl.ds(j * BATCH, BATCH)],
                    gsem[b],
                ).wait()

        def start_idx_load(b, ci):
            pltpu.async_copy(
                idx_hbm.at[pl.ds(batch0 + ci * S, S)], idx_v.at[b], isem[b]
            )

        def wait_idx(b):
            pltpu.make_async_copy(
                idx_hbm.at[pl.ds(batch0, S)], idx_v.at[b], isem[b]
            ).wait()

        def start_write(b, ci):
            pltpu.async_copy(
                rows_v.at[b], out_hbm.at[pl.ds(row0 + ci * CHUNK, CHUNK)],
                wsem[b],
            )

        def wait_write(b):
            pltpu.make_async_copy(
                rows_v.at[b], out_hbm.at[pl.ds(row0, CHUNK)], wsem[b]
            ).wait()

        def scale(b):
            def scale_body(r, carry):
                base = r * RU
                for rr in range(RU):
                    for c in range(D // LANES):
                        sl = (b, base + rr, pl.ds(c * LANES, LANES))
                        rows_v[sl] = rows_v[sl] * SCALE
                return carry

            lax.fori_loop(0, CHUNK // RU, scale_body, 0)

        def step(b, ci):
            nb = 1 - b

            @pl.when(ci + 1 < n_chunks)
            def _():
                @pl.when(ci >= 1)
                def _():
                    wait_write(nb)

                wait_idx(nb)
                fire_gathers(nb, ci + 1)

            wait_gathers(b)

            @pl.when(ci + 2 < n_chunks)
            def _():
                start_idx_load(b, ci + 2)

            scale(b)
            start_write(b, ci)

        pltpu.sync_copy(idx_hbm.at[pl.ds(batch0, S)], idx_v.at[0])
        fire_gathers(0, 0)
        start_idx_load(1, 1)

        def loop_body(g, carry):
            step(0, 2 * g)
            step(1, 2 * g + 1)
            return carry

        lax.fori_loop(0, n_chunks // 2, loop_body, 0)

        wait_write(0)
        wait_write(1)

    return body


def kernel(x, table):
    b, t = x.shape
    n_rows = b * t
    idx2d = x.astype(jnp.int32).reshape(n_rows // BATCH, BATCH)
    out = _build(n_rows)(idx2d, table)
    return out.reshape(b, t, D)
